# Initial kernel scaffold; baseline (speedup 1.0000x reference)
import jax
import jax.numpy as jnp
from jax.experimental import pallas as pl


def kernel(x, edge_index, edge_attr, Wl1, bl1, Wr1, br1, We1, att1, bias1, g1, b1, Wl2, bl2, Wr2, br2, We2, att2, bias2, g2, b2):
    raise NotImplementedError("write your pallas kernel here")


# speedup vs baseline: 12.3799x; 12.3799x over previous
import functools

import jax
import jax.numpy as jnp
from jax import lax
from jax.experimental import pallas as pl
from jax.experimental.pallas import tpu as pltpu
from jax.experimental.pallas import tpu_sc as plsc

H1, C1 = 4, 16
H2, C2 = 1, 64
NEG = 0.2
NC, NS, LANES = 2, 16, 16
NW = NC * NS
B_EDGE = 128


def _mm_t(a, w):
    return lax.dot_general(a, w, (((1,), (1,)), ((), ())),
                           preferred_element_type=jnp.float32)


def _node_proj_body(x_ref, wl_ref, bl_ref, wr_ref, br_ref, xl_ref, xr_ref):
    xb = x_ref[...]
    xl_ref[...] = _mm_t(xb, wl_ref[...]) + bl_ref[...]
    xr_ref[...] = _mm_t(xb, wr_ref[...]) + br_ref[...]


def _edge_proj_body(ea_ref, we1_ref, we2_ref, ep1_ref, ep2_ref):
    ea = ea_ref[...]
    ep1_ref[...] = _mm_t(ea, we1_ref[...])
    ep2_ref[...] = _mm_t(ea, we2_ref[...])


def _lrelu(m):
    return jnp.maximum(m, NEG * m)


def _elu(h):
    return jnp.where(h > 0.0, h, jnp.exp(jnp.minimum(h, 0.0)) - 1.0)


def _ln_act(h, g_ref, b_ref):
    mu = jnp.mean(h, axis=-1, keepdims=True)
    var = jnp.mean((h - mu) * (h - mu), axis=-1, keepdims=True)
    h = (h - mu) * lax.rsqrt(var + 1e-5) * g_ref[...] + b_ref[...]
    return _elu(h)


def _epi1_body(xl_ref, xr_ref, num_ref, asum_ref, we1_ref, att_ref, p_ref,
               bias_ref, g_ref, b_ref, wl2_ref, bl2_ref, wr2_ref, br2_ref,
               xl2_ref, xr2_ref):
    asum = asum_ref[0] + asum_ref[1]
    attr = asum[:, 0:16]
    deg = asum[:, 16:17]
    loop_attr = attr / jnp.maximum(deg, 1.0)
    epl = _mm_t(loop_attr, we1_ref[...])
    xl = xl_ref[...]
    xr = xr_ref[...]
    m = _lrelu(xl + xr + epl)
    p = p_ref[...]
    a_self = lax.dot_general(m * att_ref[...], p, (((1,), (0,)), ((), ())),
                             preferred_element_type=jnp.float32)
    w_self = jnp.exp(a_self)
    wfull = lax.dot_general(w_self, p, (((1,), (1,)), ((), ())),
                            preferred_element_type=jnp.float32)
    nump = num_ref[0] + num_ref[1]
    num = nump[:, 0:64] + xl * wfull
    den_h = nump[:, 64:64 + H1] + w_self
    den = lax.dot_general(den_h, p, (((1,), (1,)), ((), ())),
                          preferred_element_type=jnp.float32)
    out = num / den + bias_ref[...]
    h = _ln_act(out, g_ref, b_ref)
    xl2_ref[...] = _mm_t(h, wl2_ref[...]) + bl2_ref[...]
    xr2_ref[...] = _mm_t(h, wr2_ref[...]) + br2_ref[...]


def _epi2_body(xl_ref, xr_ref, num_ref, asum_ref, we2_ref, att_ref,
               bias_ref, g_ref, b_ref, out_ref):
    asum = asum_ref[0] + asum_ref[1]
    attr = asum[:, 0:16]
    deg = asum[:, 16:17]
    loop_attr = attr / jnp.maximum(deg, 1.0)
    epl = _mm_t(loop_attr, we2_ref[...])
    xl = xl_ref[...]
    xr = xr_ref[...]
    m = _lrelu(xl + xr + epl)
    a_self = jnp.sum(m * att_ref[...], axis=-1, keepdims=True)
    w_self = jnp.exp(a_self)
    nump = num_ref[0] + num_ref[1]
    num = nump[:, 0:64] + xl * w_self
    den = nump[:, 64:65] + w_self
    out = num / den + bias_ref[...]
    out_ref[...] = _ln_act(out, g_ref, b_ref)


def _zero_stage(stage, rows, cols):
    z = jnp.zeros((LANES,), jnp.float32)

    def zr(r, _):
        for k in range(cols // LANES):
            stage[r, pl.ds(k * LANES, LANES)] = z
        return 0

    lax.fori_loop(0, rows, zr, 0)


def _zero_shared_rows(stage, accum, base, rows):
    off = 0
    while off < rows:
        c = min(B_EDGE, rows - off)
        pltpu.sync_copy(stage.at[pl.ds(0, c)], accum.at[pl.ds(base + off, c)])
        off += c


def _export_shared_rows(accum, out, cid, base, rows):
    off = 0
    while off < rows:
        c = min(B_EDGE, rows - off)
        pltpu.sync_copy(accum.at[pl.ds(base + off, c)],
                        out.at[cid, pl.ds(base + off, c)])
        off += c


def _make_attr_sum_kernel(nacc, e_pad):
    epw = e_pad // NW
    nb = epw // B_EDGE
    rows_pt = nacc // NS
    mesh = plsc.VectorSubcoreMesh(core_axis_name="c", subcore_axis_name="s",
                                  num_cores=NC, num_subcores=NS)

    @functools.partial(
        pl.kernel,
        out_type=jax.ShapeDtypeStruct((NC, nacc, 32), jnp.float32),
        mesh=mesh,
        compiler_params=pltpu.CompilerParams(needs_layout_passes=False, use_tc_tiling_on_sc=False),
        scratch_types=[
            pltpu.VMEM_SHARED((nacc, 32), jnp.float32),
            pltpu.VMEM((B_EDGE,), jnp.int32),
            pltpu.VMEM((B_EDGE, 16), jnp.float32),
            pltpu.VMEM((B_EDGE, 32), jnp.float32),
        ],
    )
    def body(ea_hbm, dst_hbm, out_hbm, accum, idx_d, ea_v, stage):
        cid = lax.axis_index("c")
        sid = lax.axis_index("s")
        wid = cid * NS + sid
        _zero_stage(stage, B_EDGE, 32)
        _zero_shared_rows(stage, accum, sid * rows_pt, rows_pt)
        ones = jnp.ones((LANES,), jnp.float32)
        c16 = jnp.full((LANES,), 16, jnp.int32)

        def gset(j, _):
            ev = lax.iota(jnp.int32, LANES) + j * LANES
            plsc.store_scatter(stage, [ev, c16], ones)
            return 0

        lax.fori_loop(0, B_EDGE // LANES, gset, 0)
        plsc.subcore_barrier()

        def batch(b, _):
            e0 = wid * epw + b * B_EDGE
            pltpu.sync_copy(dst_hbm.at[pl.ds(e0, B_EDGE)], idx_d)
            pltpu.sync_copy(ea_hbm.at[pl.ds(e0, B_EDGE)], ea_v)

            def group(j, _):
                ev = lax.iota(jnp.int32, LANES) + j * LANES
                for c in range(16):
                    cv = jnp.full((LANES,), c, jnp.int32)
                    g = plsc.load_gather(ea_v, [ev, cv])
                    plsc.store_scatter(stage, [ev, cv], g)
                return 0

            lax.fori_loop(0, B_EDGE // LANES, group, 0)
            pltpu.sync_copy(stage, accum.at[idx_d], add=True)
            return 0

        lax.fori_loop(0, nb, batch, 0)
        plsc.subcore_barrier()
        _export_shared_rows(accum, out_hbm, cid, sid * rows_pt, rows_pt)

    return body


def _make_edge_kernel(h_heads, c_dim, nacc, e_pad):
    epw = e_pad // NW
    nb = epw // B_EDGE
    rows_pt = nacc // NS
    keep_xl = c_dim <= 16
    mesh = plsc.VectorSubcoreMesh(core_axis_name="c", subcore_axis_name="s",
                                  num_cores=NC, num_subcores=NS)

    @functools.partial(
        pl.kernel,
        out_type=jax.ShapeDtypeStruct((NC, nacc, 80), jnp.float32),
        mesh=mesh,
        compiler_params=pltpu.CompilerParams(needs_layout_passes=False, use_tc_tiling_on_sc=False),
        scratch_types=[
            pltpu.VMEM_SHARED((nacc, 80), jnp.float32),
            pltpu.VMEM((B_EDGE,), jnp.int32),
            pltpu.VMEM((B_EDGE,), jnp.int32),
            pltpu.VMEM((B_EDGE, 64), jnp.float32),
            pltpu.VMEM((B_EDGE, 64), jnp.float32),
            pltpu.VMEM((B_EDGE, 64), jnp.float32),
            pltpu.VMEM((B_EDGE, 80), jnp.float32),
            pltpu.VMEM((80,), jnp.float32),
            pltpu.SemaphoreType.DMA,
            pltpu.SemaphoreType.DMA,
            pltpu.SemaphoreType.DMA,
        ],
    )
    def body(xl_hbm, xr_hbm, ep_hbm, src_hbm, dst_hbm, att_hbm, out_hbm,
             accum, idx_s, idx_d, xl_v, xr_v, ep_v, stage, att_v,
             sem1, sem2, sem3):
        cid = lax.axis_index("c")
        sid = lax.axis_index("s")
        wid = cid * NS + sid
        _zero_stage(stage, B_EDGE, 80)
        _zero_shared_rows(stage, accum, sid * rows_pt, rows_pt)
        pltpu.sync_copy(att_hbm, att_v.at[pl.ds(16, 64)])
        plsc.subcore_barrier()

        def batch(b, _):
            e0 = wid * epw + b * B_EDGE
            pltpu.sync_copy(src_hbm.at[pl.ds(e0, B_EDGE)], idx_s)
            pltpu.sync_copy(dst_hbm.at[pl.ds(e0, B_EDGE)], idx_d)
            cp1 = pltpu.async_copy(xl_hbm.at[idx_s], xl_v, sem1)
            cp2 = pltpu.async_copy(xr_hbm.at[idx_d], xr_v, sem2)
            cp3 = pltpu.async_copy(ep_hbm.at[pl.ds(e0, B_EDGE)], ep_v, sem3)
            cp1.wait()
            cp2.wait()
            cp3.wait()

            def group(j, _):
                ev = lax.iota(jnp.int32, LANES) + j * LANES
                for h in range(h_heads):
                    a = None
                    xls = []
                    for cl in range(c_dim):
                        c = h * c_dim + cl
                        cv = jnp.full((LANES,), c, jnp.int32)
                        xlc = plsc.load_gather(xl_v, [ev, cv])
                        xrc = plsc.load_gather(xr_v, [ev, cv])
                        epc = plsc.load_gather(ep_v, [ev, cv])
                        u = _lrelu(xlc + xrc + epc)
                        attc = plsc.load_gather(
                            att_v, [jnp.full((LANES,), 16 + c, jnp.int32)])
                        t = u * attc
                        a = t if a is None else a + t
                        if keep_xl:
                            xls.append(xlc)
                    w = jnp.exp(a)
                    for cl in range(c_dim):
                        c = h * c_dim + cl
                        cv = jnp.full((LANES,), c, jnp.int32)
                        if keep_xl:
                            xlc = xls[cl]
                        else:
                            xlc = plsc.load_gather(xl_v, [ev, cv])
                        plsc.store_scatter(stage, [ev, cv], xlc * w)
                    hv = jnp.full((LANES,), 64 + h, jnp.int32)
                    plsc.store_scatter(stage, [ev, hv], w)
                return 0

            lax.fori_loop(0, B_EDGE // LANES, group, 0)
            pltpu.sync_copy(stage, accum.at[idx_d], add=True)
            return 0

        lax.fori_loop(0, nb, batch, 0)
        plsc.subcore_barrier()
        _export_shared_rows(accum, out_hbm, cid, sid * rows_pt, rows_pt)

    return body


def kernel(x, edge_index, edge_attr, Wl1, bl1, Wr1, br1, We1, att1, bias1,
           g1, b1, Wl2, bl2, Wr2, br2, We2, att2, bias2, g2, b2):
    n_nodes, d_in = x.shape
    e_edges = edge_index.shape[1]
    f32 = jnp.float32

    src = edge_index[0]
    dst = edge_index[1]

    chunk = NW * B_EDGE
    e_pad = -(-e_edges // chunk) * chunk
    nacc = -(-(n_nodes + 1) // (NS * 8)) * (NS * 8)
    if e_pad != e_edges:
        pad = e_pad - e_edges
        src = jnp.concatenate([src, jnp.zeros((pad,), src.dtype)])
        dst = jnp.concatenate([dst, jnp.full((pad,), n_nodes, dst.dtype)])
        edge_attr = jnp.concatenate(
            [edge_attr, jnp.zeros((pad, edge_attr.shape[1]), edge_attr.dtype)])

    bn = 1000
    n_pad = -(-n_nodes // bn) * bn
    if n_pad != n_nodes:
        x = jnp.concatenate([x, jnp.zeros((n_pad - n_nodes, d_in), f32)])
    n_blocks = n_pad // bn

    row2d = lambda a: a.reshape(1, -1).astype(f32)

    xl1, xr1 = pl.pallas_call(
        _node_proj_body,
        grid=(n_blocks,),
        in_specs=[
            pl.BlockSpec((bn, d_in), lambda i: (i, 0)),
            pl.BlockSpec(Wl1.shape, lambda i: (0, 0)),
            pl.BlockSpec((1, H1 * C1), lambda i: (0, 0)),
            pl.BlockSpec(Wr1.shape, lambda i: (0, 0)),
            pl.BlockSpec((1, H1 * C1), lambda i: (0, 0)),
        ],
        out_specs=[
            pl.BlockSpec((bn, H1 * C1), lambda i: (i, 0)),
            pl.BlockSpec((bn, H1 * C1), lambda i: (i, 0)),
        ],
        out_shape=[
            jax.ShapeDtypeStruct((n_pad, H1 * C1), f32),
            jax.ShapeDtypeStruct((n_pad, H1 * C1), f32),
        ],
    )(x, Wl1, row2d(bl1), Wr1, row2d(br1))

    be = 6400
    eb = e_pad // be
    ep1, ep2 = pl.pallas_call(
        _edge_proj_body,
        grid=(eb,),
        in_specs=[
            pl.BlockSpec((be, edge_attr.shape[1]), lambda i: (i, 0)),
            pl.BlockSpec(We1.shape, lambda i: (0, 0)),
            pl.BlockSpec(We2.shape, lambda i: (0, 0)),
        ],
        out_specs=[
            pl.BlockSpec((be, H1 * C1), lambda i: (i, 0)),
            pl.BlockSpec((be, H2 * C2), lambda i: (i, 0)),
        ],
        out_shape=[
            jax.ShapeDtypeStruct((e_pad, H1 * C1), f32),
            jax.ShapeDtypeStruct((e_pad, H2 * C2), f32),
        ],
    )(edge_attr.astype(f32), We1, We2)

    asum = _make_attr_sum_kernel(nacc, e_pad)(edge_attr.astype(f32), dst)

    if e_pad == e_edges or n_pad > n_nodes:
        xl1g, xr1g = xl1, xr1
    else:
        zrow = jnp.zeros((1, H1 * C1), f32)
        xl1g = jnp.concatenate([xl1, zrow])
        xr1g = jnp.concatenate([xr1, zrow])
    num1 = _make_edge_kernel(H1, C1, nacc, e_pad)(
        xl1g, xr1g, ep1, src, dst, att1.reshape(-1).astype(f32))

    pool = jnp.repeat(jnp.eye(H1, dtype=f32), C1, axis=0)
    xl2, xr2 = pl.pallas_call(
        _epi1_body,
        grid=(n_nodes // bn if n_nodes % bn == 0 else n_blocks,),
        in_specs=[
            pl.BlockSpec((bn, H1 * C1), lambda i: (i, 0)),
            pl.BlockSpec((bn, H1 * C1), lambda i: (i, 0)),
            pl.BlockSpec((NC, bn, 80), lambda i: (0, i, 0)),
            pl.BlockSpec((NC, bn, 32), lambda i: (0, i, 0)),
            pl.BlockSpec(We1.shape, lambda i: (0, 0)),
            pl.BlockSpec((1, H1 * C1), lambda i: (0, 0)),
            pl.BlockSpec((H1 * C1, H1), lambda i: (0, 0)),
            pl.BlockSpec((1, H1 * C1), lambda i: (0, 0)),
            pl.BlockSpec((1, H1 * C1), lambda i: (0, 0)),
            pl.BlockSpec((1, H1 * C1), lambda i: (0, 0)),
            pl.BlockSpec(Wl2.shape, lambda i: (0, 0)),
            pl.BlockSpec((1, H2 * C2), lambda i: (0, 0)),
            pl.BlockSpec(Wr2.shape, lambda i: (0, 0)),
            pl.BlockSpec((1, H2 * C2), lambda i: (0, 0)),
        ],
        out_specs=[
            pl.BlockSpec((bn, H2 * C2), lambda i: (i, 0)),
            pl.BlockSpec((bn, H2 * C2), lambda i: (i, 0)),
        ],
        out_shape=[
            jax.ShapeDtypeStruct((n_pad, H2 * C2), f32),
            jax.ShapeDtypeStruct((n_pad, H2 * C2), f32),
        ],
    )(xl1[:n_pad], xr1[:n_pad], num1, asum, We1,
      row2d(att1.reshape(-1)), pool, row2d(bias1), row2d(g1), row2d(b1),
      Wl2, row2d(bl2), Wr2, row2d(br2))

    if e_pad == e_edges or n_pad > n_nodes:
        xl2g, xr2g = xl2, xr2
    else:
        zrow = jnp.zeros((1, H2 * C2), f32)
        xl2g = jnp.concatenate([xl2, zrow])
        xr2g = jnp.concatenate([xr2, zrow])
    num2 = _make_edge_kernel(H2, C2, nacc, e_pad)(
        xl2g, xr2g, ep2, src, dst, att2.reshape(-1).astype(f32))

    out = pl.pallas_call(
        _epi2_body,
        grid=(n_blocks,),
        in_specs=[
            pl.BlockSpec((bn, H2 * C2), lambda i: (i, 0)),
            pl.BlockSpec((bn, H2 * C2), lambda i: (i, 0)),
            pl.BlockSpec((NC, bn, 80), lambda i: (0, i, 0)),
            pl.BlockSpec((NC, bn, 32), lambda i: (0, i, 0)),
            pl.BlockSpec(We2.shape, lambda i: (0, 0)),
            pl.BlockSpec((1, H2 * C2), lambda i: (0, 0)),
            pl.BlockSpec((1, H2 * C2), lambda i: (0, 0)),
            pl.BlockSpec((1, H2 * C2), lambda i: (0, 0)),
            pl.BlockSpec((1, H2 * C2), lambda i: (0, 0)),
        ],
        out_specs=pl.BlockSpec((bn, H2 * C2), lambda i: (i, 0)),
        out_shape=jax.ShapeDtypeStruct((n_pad, H2 * C2), f32),
    )(xl2[:n_pad], xr2[:n_pad], num2, asum, We2,
      row2d(att2.reshape(-1)), row2d(bias2), row2d(g2), row2d(b2))

    return out[:n_nodes]


# speedup vs baseline: 14.1345x; 1.1417x over previous
import functools

import jax
import jax.numpy as jnp
from jax import lax
from jax.experimental import pallas as pl
from jax.experimental.pallas import tpu as pltpu
from jax.experimental.pallas import tpu_sc as plsc

H1, C1 = 4, 16
H2, C2 = 1, 64
NEG = 0.2
NC, NS, LANES = 2, 16, 16
NW = NC * NS
B_EDGE = 128
IDX_CHUNK = 8


def _mm_t(a, w):
    return lax.dot_general(a, w, (((1,), (1,)), ((), ())),
                           preferred_element_type=jnp.float32)


def _node_proj_body(x_ref, wl_ref, bl_ref, wr_ref, br_ref, xl_ref, xr_ref):
    xb = x_ref[...]
    xl_ref[...] = _mm_t(xb, wl_ref[...]) + bl_ref[...]
    xr_ref[...] = _mm_t(xb, wr_ref[...]) + br_ref[...]


def _edge_proj_body(ea_ref, we1_ref, we2_ref, ep1_ref, ep2_ref):
    ea = ea_ref[...]
    ep1_ref[...] = _mm_t(ea, we1_ref[...])
    ep2_ref[...] = _mm_t(ea, we2_ref[...])


def _lrelu(m):
    return jnp.maximum(m, NEG * m)


def _elu(h):
    return jnp.where(h > 0.0, h, jnp.exp(jnp.minimum(h, 0.0)) - 1.0)


def _ln_act(h, g_ref, b_ref):
    mu = jnp.mean(h, axis=-1, keepdims=True)
    var = jnp.mean((h - mu) * (h - mu), axis=-1, keepdims=True)
    h = (h - mu) * lax.rsqrt(var + 1e-5) * g_ref[...] + b_ref[...]
    return _elu(h)


def _epi1_body(xl_ref, xr_ref, num_ref, asum_ref, we1_ref, att_ref, p_ref,
               bias_ref, g_ref, b_ref, wl2_ref, bl2_ref, wr2_ref, br2_ref,
               xl2_ref, xr2_ref):
    asum = asum_ref[0] + asum_ref[1]
    attr = asum[:, 0:16]
    deg = asum[:, 16:17]
    loop_attr = attr / jnp.maximum(deg, 1.0)
    epl = _mm_t(loop_attr, we1_ref[...])
    xl = xl_ref[...]
    xr = xr_ref[...]
    m = _lrelu(xl + xr + epl)
    p = p_ref[...]
    a_self = lax.dot_general(m * att_ref[...], p, (((1,), (0,)), ((), ())),
                             preferred_element_type=jnp.float32)
    w_self = jnp.exp(a_self)
    wfull = lax.dot_general(w_self, p, (((1,), (1,)), ((), ())),
                            preferred_element_type=jnp.float32)
    nump = num_ref[0] + num_ref[1]
    num = nump[:, 0:64] + xl * wfull
    den_h = nump[:, 64:64 + H1] + w_self
    den = lax.dot_general(den_h, p, (((1,), (1,)), ((), ())),
                          preferred_element_type=jnp.float32)
    out = num / den + bias_ref[...]
    h = _ln_act(out, g_ref, b_ref)
    xl2_ref[...] = _mm_t(h, wl2_ref[...]) + bl2_ref[...]
    xr2_ref[...] = _mm_t(h, wr2_ref[...]) + br2_ref[...]


def _epi2_body(xl_ref, xr_ref, num_ref, asum_ref, we2_ref, att_ref,
               bias_ref, g_ref, b_ref, out_ref):
    asum = asum_ref[0] + asum_ref[1]
    attr = asum[:, 0:16]
    deg = asum[:, 16:17]
    loop_attr = attr / jnp.maximum(deg, 1.0)
    epl = _mm_t(loop_attr, we2_ref[...])
    xl = xl_ref[...]
    xr = xr_ref[...]
    m = _lrelu(xl + xr + epl)
    a_self = jnp.sum(m * att_ref[...], axis=-1, keepdims=True)
    w_self = jnp.exp(a_self)
    nump = num_ref[0] + num_ref[1]
    num = nump[:, 0:64] + xl * w_self
    den = nump[:, 64:65] + w_self
    out = num / den + bias_ref[...]
    out_ref[...] = _ln_act(out, g_ref, b_ref)


def _zero_stage(stage, rows, cols):
    z = jnp.zeros((LANES,), jnp.float32)

    def zr(r, _):
        for k in range(cols // LANES):
            stage[r, pl.ds(k * LANES, LANES)] = z
        return 0

    lax.fori_loop(0, rows, zr, 0)


def _zero_shared_rows(stage, accum, base, rows):
    off = 0
    while off < rows:
        c = min(B_EDGE, rows - off)
        pltpu.sync_copy(stage.at[pl.ds(0, c)], accum.at[pl.ds(base + off, c)])
        off += c


def _export_shared_rows(accum, out, cid, base, rows):
    off = 0
    while off < rows:
        c = min(B_EDGE, rows - off)
        pltpu.sync_copy(accum.at[pl.ds(base + off, c)],
                        out.at[cid, pl.ds(base + off, c)])
        off += c


def _make_attr_sum_kernel(nacc, e_pad):
    epw = e_pad // NW
    nb = epw // B_EDGE
    rows_pt = nacc // NS
    mesh = plsc.VectorSubcoreMesh(core_axis_name="c", subcore_axis_name="s",
                                  num_cores=NC, num_subcores=NS)

    @functools.partial(
        pl.kernel,
        out_type=jax.ShapeDtypeStruct((NC, nacc, 32), jnp.float32),
        mesh=mesh,
        compiler_params=pltpu.CompilerParams(needs_layout_passes=False, use_tc_tiling_on_sc=False),
        scratch_types=[
            pltpu.VMEM_SHARED((nacc, 32), jnp.float32),
            pltpu.VMEM((B_EDGE,), jnp.int32),
            pltpu.VMEM((B_EDGE, 16), jnp.float32),
            pltpu.VMEM((B_EDGE, 32), jnp.float32),
        ],
    )
    def body(ea_hbm, dst_hbm, out_hbm, accum, idx_d, ea_v, stage):
        cid = lax.axis_index("c")
        sid = lax.axis_index("s")
        wid = cid * NS + sid
        _zero_stage(stage, B_EDGE, 32)
        _zero_shared_rows(stage, accum, sid * rows_pt, rows_pt)
        ones = jnp.ones((LANES,), jnp.float32)
        c16 = jnp.full((LANES,), 16, jnp.int32)

        def gset(j, _):
            ev = lax.iota(jnp.int32, LANES) + j * LANES
            plsc.store_scatter(stage, [ev, c16], ones)
            return 0

        lax.fori_loop(0, B_EDGE // LANES, gset, 0)
        plsc.subcore_barrier()

        def batch(b, _):
            e0 = wid * epw + b * B_EDGE
            pltpu.sync_copy(dst_hbm.at[pl.ds(e0, B_EDGE)], idx_d)
            pltpu.sync_copy(ea_hbm.at[pl.ds(e0, B_EDGE)], ea_v)

            def group(j, _):
                ev = lax.iota(jnp.int32, LANES) + j * LANES
                for c in range(16):
                    cv = jnp.full((LANES,), c, jnp.int32)
                    g = plsc.load_gather(ea_v, [ev, cv])
                    plsc.store_scatter(stage, [ev, cv], g)
                return 0

            lax.fori_loop(0, B_EDGE // LANES, group, 0)
            pltpu.sync_copy(stage, accum.at[idx_d], add=True)
            return 0

        lax.fori_loop(0, nb, batch, 0)
        plsc.subcore_barrier()
        _export_shared_rows(accum, out_hbm, cid, sid * rows_pt, rows_pt)

    return body


def _make_edge_kernel(h_heads, c_dim, nacc, e_pad):
    epw = e_pad // NW
    nb = epw // B_EDGE
    assert nb % 2 == 0 and nb % IDX_CHUNK == 0
    rows_pt = nacc // NS
    keep_xl = c_dim <= 16
    mesh = plsc.VectorSubcoreMesh(core_axis_name="c", subcore_axis_name="s",
                                  num_cores=NC, num_subcores=NS)

    @functools.partial(
        pl.kernel,
        out_type=jax.ShapeDtypeStruct((NC, nacc, 80), jnp.float32),
        mesh=mesh,
        compiler_params=pltpu.CompilerParams(needs_layout_passes=False, use_tc_tiling_on_sc=False),
        scratch_types=[
            pltpu.VMEM_SHARED((nacc, 80), jnp.float32),
            pltpu.VMEM((IDX_CHUNK, B_EDGE), jnp.int32),
            pltpu.VMEM((IDX_CHUNK, B_EDGE), jnp.int32),
            pltpu.VMEM((IDX_CHUNK, B_EDGE), jnp.int32),
            pltpu.VMEM((IDX_CHUNK, B_EDGE), jnp.int32),
            pltpu.VMEM((B_EDGE, 64), jnp.float32),
            pltpu.VMEM((B_EDGE, 64), jnp.float32),
            pltpu.VMEM((B_EDGE, 64), jnp.float32),
            pltpu.VMEM((B_EDGE, 64), jnp.float32),
            pltpu.VMEM((B_EDGE, 64), jnp.float32),
            pltpu.VMEM((B_EDGE, 64), jnp.float32),
            pltpu.VMEM((B_EDGE, 80), jnp.float32),
            pltpu.VMEM((B_EDGE, 80), jnp.float32),
            pltpu.VMEM((80,), jnp.float32),
            pltpu.SemaphoreType.DMA,
            pltpu.SemaphoreType.DMA,
            pltpu.SemaphoreType.DMA,
            pltpu.SemaphoreType.DMA,
            pltpu.SemaphoreType.DMA,
            pltpu.SemaphoreType.DMA,
            pltpu.SemaphoreType.DMA,
            pltpu.SemaphoreType.DMA,
        ],
    )
    def body(xl_hbm, xr_hbm, ep_hbm, src_hbm, dst_hbm, att_hbm, out_hbm,
             accum, ixs0, ixs1, ixd0, ixd1,
             xl_a, xl_b, xr_a, xr_b, ep_a, ep_b, st_a, st_b, att_v,
             gx0, gx1, gr0, gr1, ge0, ge1, ss0, ss1):
        cid = lax.axis_index("c")
        sid = lax.axis_index("s")
        wid = cid * NS + sid
        xl_s = (xl_a, xl_b)
        xr_s = (xr_a, xr_b)
        ep_s = (ep_a, ep_b)
        st_s = (st_a, st_b)
        ixs = (ixs0, ixs1)
        ixd = (ixd0, ixd1)
        gx = (gx0, gx1)
        gr = (gr0, gr1)
        ge = (ge0, ge1)
        ss = (ss0, ss1)
        _zero_stage(st_a, B_EDGE, 80)
        _zero_stage(st_b, B_EDGE, 80)
        _zero_shared_rows(st_a, accum, sid * rows_pt, rows_pt)
        pltpu.sync_copy(att_hbm, att_v.at[pl.ds(16, 64)])
        plsc.subcore_barrier()

        def fetch_chunk(c, par):
            pltpu.sync_copy(src_hbm.at[wid, pl.ds(c * IDX_CHUNK, IDX_CHUNK)],
                            ixs[par])
            pltpu.sync_copy(dst_hbm.at[wid, pl.ds(c * IDX_CHUNK, IDX_CHUNK)],
                            ixd[par])

        def issue_gathers(bi, slot, par):
            r = bi % IDX_CHUNK
            pltpu.async_copy(xl_hbm.at[ixs[par].at[r]], xl_s[slot], gx[slot])
            pltpu.async_copy(xr_hbm.at[ixd[par].at[r]], xr_s[slot], gr[slot])

        def compute(slot):
            xlv, xrv, epv, stage = xl_s[slot], xr_s[slot], ep_s[slot], st_s[slot]

            def group(j, _):
                ev = lax.iota(jnp.int32, LANES) + j * LANES
                for h in range(h_heads):
                    a = None
                    xls = []
                    for cl in range(c_dim):
                        c = h * c_dim + cl
                        cv = jnp.full((LANES,), c, jnp.int32)
                        xlc = plsc.load_gather(xlv, [ev, cv])
                        xrc = plsc.load_gather(xrv, [ev, cv])
                        epc = plsc.load_gather(epv, [ev, cv])
                        u = _lrelu(xlc + xrc + epc)
                        attc = plsc.load_gather(
                            att_v, [jnp.full((LANES,), 16 + c, jnp.int32)])
                        t = u * attc
                        a = t if a is None else a + t
                        if keep_xl:
                            xls.append(xlc)
                    w = jnp.exp(a)
                    for cl in range(c_dim):
                        c = h * c_dim + cl
                        cv = jnp.full((LANES,), c, jnp.int32)
                        if keep_xl:
                            xlc = xls[cl]
                        else:
                            xlc = plsc.load_gather(xlv, [ev, cv])
                        plsc.store_scatter(stage, [ev, cv], xlc * w)
                    hv = jnp.full((LANES,), 64 + h, jnp.int32)
                    plsc.store_scatter(stage, [ev, hv], w)
                return 0

            lax.fori_loop(0, B_EDGE // LANES, group, 0)

        def wait_gathers(slot):
            pltpu.make_async_copy(xl_hbm.at[ixs[0].at[0]], xl_s[slot],
                                  gx[slot]).wait()
            pltpu.make_async_copy(xr_hbm.at[ixd[0].at[0]], xr_s[slot],
                                  gr[slot]).wait()
            pltpu.make_async_copy(ep_hbm.at[pl.ds(0, B_EDGE)], ep_s[slot],
                                  ge[slot]).wait()

        def wait_scatter(slot):
            pltpu.make_async_copy(st_s[slot], accum.at[ixd[0].at[0]],
                                  ss[slot]).wait()

        def phase(i, slot):
            other = 1 - slot
            @pl.when(i >= 2)
            def _():
                wait_scatter(slot)

            wait_gathers(slot)
            nxt = i + 1

            @pl.when(nxt < nb)
            def _():
                cpar = (nxt // IDX_CHUNK) % 2
                refresh = (nxt % IDX_CHUNK) == 0

                @pl.when(jnp.logical_and(refresh, cpar == 0))
                def _():
                    fetch_chunk(nxt // IDX_CHUNK, 0)

                @pl.when(jnp.logical_and(refresh, cpar == 1))
                def _():
                    fetch_chunk(nxt // IDX_CHUNK, 1)

                @pl.when(cpar == 0)
                def _():
                    issue_gathers(nxt, other, 0)

                @pl.when(cpar == 1)
                def _():
                    issue_gathers(nxt, other, 1)

                pltpu.async_copy(
                    ep_hbm.at[pl.ds(wid * epw + nxt * B_EDGE, B_EDGE)],
                    ep_s[other], ge[other])

            compute(slot)
            rcur = i % IDX_CHUNK
            ccur = (i // IDX_CHUNK) % 2

            @pl.when(ccur == 0)
            def _():
                pltpu.async_copy(st_s[slot], accum.at[ixd[0].at[rcur]],
                                 ss[slot], add=True)

            @pl.when(ccur == 1)
            def _():
                pltpu.async_copy(st_s[slot], accum.at[ixd[1].at[rcur]],
                                 ss[slot], add=True)

        fetch_chunk(0, 0)
        issue_gathers(0, 0, 0)
        pltpu.async_copy(ep_hbm.at[pl.ds(wid * epw, B_EDGE)], ep_a, ge0)

        def pair(k, _):
            phase(2 * k, 0)
            phase(2 * k + 1, 1)
            return 0

        lax.fori_loop(0, nb // 2, pair, 0)
        wait_scatter(0)
        wait_scatter(1)
        plsc.subcore_barrier()
        _export_shared_rows(accum, out_hbm, cid, sid * rows_pt, rows_pt)

    return body


def kernel(x, edge_index, edge_attr, Wl1, bl1, Wr1, br1, We1, att1, bias1,
           g1, b1, Wl2, bl2, Wr2, br2, We2, att2, bias2, g2, b2):
    n_nodes, d_in = x.shape
    e_edges = edge_index.shape[1]
    f32 = jnp.float32

    src = edge_index[0]
    dst = edge_index[1]

    chunk = NW * B_EDGE * IDX_CHUNK
    e_pad = -(-e_edges // chunk) * chunk
    nacc = -(-(n_nodes + 1) // (NS * 8)) * (NS * 8)
    if e_pad != e_edges:
        pad = e_pad - e_edges
        src = jnp.concatenate([src, jnp.zeros((pad,), src.dtype)])
        dst = jnp.concatenate([dst, jnp.full((pad,), n_nodes, dst.dtype)])
        edge_attr = jnp.concatenate(
            [edge_attr, jnp.zeros((pad, edge_attr.shape[1]), edge_attr.dtype)])

    bn = 1000
    n_pad = -(-n_nodes // bn) * bn
    if n_pad != n_nodes:
        x = jnp.concatenate([x, jnp.zeros((n_pad - n_nodes, d_in), f32)])
    n_blocks = n_pad // bn

    row2d = lambda a: a.reshape(1, -1).astype(f32)

    xl1, xr1 = pl.pallas_call(
        _node_proj_body,
        grid=(n_blocks,),
        in_specs=[
            pl.BlockSpec((bn, d_in), lambda i: (i, 0)),
            pl.BlockSpec(Wl1.shape, lambda i: (0, 0)),
            pl.BlockSpec((1, H1 * C1), lambda i: (0, 0)),
            pl.BlockSpec(Wr1.shape, lambda i: (0, 0)),
            pl.BlockSpec((1, H1 * C1), lambda i: (0, 0)),
        ],
        out_specs=[
            pl.BlockSpec((bn, H1 * C1), lambda i: (i, 0)),
            pl.BlockSpec((bn, H1 * C1), lambda i: (i, 0)),
        ],
        out_shape=[
            jax.ShapeDtypeStruct((n_pad, H1 * C1), f32),
            jax.ShapeDtypeStruct((n_pad, H1 * C1), f32),
        ],
    )(x, Wl1, row2d(bl1), Wr1, row2d(br1))

    be = 6400
    eb = e_pad // be
    ep1, ep2 = pl.pallas_call(
        _edge_proj_body,
        grid=(eb,),
        in_specs=[
            pl.BlockSpec((be, edge_attr.shape[1]), lambda i: (i, 0)),
            pl.BlockSpec(We1.shape, lambda i: (0, 0)),
            pl.BlockSpec(We2.shape, lambda i: (0, 0)),
        ],
        out_specs=[
            pl.BlockSpec((be, H1 * C1), lambda i: (i, 0)),
            pl.BlockSpec((be, H2 * C2), lambda i: (i, 0)),
        ],
        out_shape=[
            jax.ShapeDtypeStruct((e_pad, H1 * C1), f32),
            jax.ShapeDtypeStruct((e_pad, H2 * C2), f32),
        ],
    )(edge_attr.astype(f32), We1, We2)

    asum = _make_attr_sum_kernel(nacc, e_pad)(edge_attr.astype(f32), dst)

    if e_pad == e_edges or n_pad > n_nodes:
        xl1g, xr1g = xl1, xr1
    else:
        zrow = jnp.zeros((1, H1 * C1), f32)
        xl1g = jnp.concatenate([xl1, zrow])
        xr1g = jnp.concatenate([xr1, zrow])
    nb_w = e_pad // NW // B_EDGE
    src3 = src.reshape(NW, nb_w, B_EDGE)
    dst3 = dst.reshape(NW, nb_w, B_EDGE)
    num1 = _make_edge_kernel(H1, C1, nacc, e_pad)(
        xl1g, xr1g, ep1, src3, dst3, att1.reshape(-1).astype(f32))

    pool = jnp.repeat(jnp.eye(H1, dtype=f32), C1, axis=0)
    xl2, xr2 = pl.pallas_call(
        _epi1_body,
        grid=(n_nodes // bn if n_nodes % bn == 0 else n_blocks,),
        in_specs=[
            pl.BlockSpec((bn, H1 * C1), lambda i: (i, 0)),
            pl.BlockSpec((bn, H1 * C1), lambda i: (i, 0)),
            pl.BlockSpec((NC, bn, 80), lambda i: (0, i, 0)),
            pl.BlockSpec((NC, bn, 32), lambda i: (0, i, 0)),
            pl.BlockSpec(We1.shape, lambda i: (0, 0)),
            pl.BlockSpec((1, H1 * C1), lambda i: (0, 0)),
            pl.BlockSpec((H1 * C1, H1), lambda i: (0, 0)),
            pl.BlockSpec((1, H1 * C1), lambda i: (0, 0)),
            pl.BlockSpec((1, H1 * C1), lambda i: (0, 0)),
            pl.BlockSpec((1, H1 * C1), lambda i: (0, 0)),
            pl.BlockSpec(Wl2.shape, lambda i: (0, 0)),
            pl.BlockSpec((1, H2 * C2), lambda i: (0, 0)),
            pl.BlockSpec(Wr2.shape, lambda i: (0, 0)),
            pl.BlockSpec((1, H2 * C2), lambda i: (0, 0)),
        ],
        out_specs=[
            pl.BlockSpec((bn, H2 * C2), lambda i: (i, 0)),
            pl.BlockSpec((bn, H2 * C2), lambda i: (i, 0)),
        ],
        out_shape=[
            jax.ShapeDtypeStruct((n_pad, H2 * C2), f32),
            jax.ShapeDtypeStruct((n_pad, H2 * C2), f32),
        ],
    )(xl1[:n_pad], xr1[:n_pad], num1, asum, We1,
      row2d(att1.reshape(-1)), pool, row2d(bias1), row2d(g1), row2d(b1),
      Wl2, row2d(bl2), Wr2, row2d(br2))

    if e_pad == e_edges or n_pad > n_nodes:
        xl2g, xr2g = xl2, xr2
    else:
        zrow = jnp.zeros((1, H2 * C2), f32)
        xl2g = jnp.concatenate([xl2, zrow])
        xr2g = jnp.concatenate([xr2, zrow])
    num2 = _make_edge_kernel(H2, C2, nacc, e_pad)(
        xl2g, xr2g, ep2, src3, dst3, att2.reshape(-1).astype(f32))

    out = pl.pallas_call(
        _epi2_body,
        grid=(n_blocks,),
        in_specs=[
            pl.BlockSpec((bn, H2 * C2), lambda i: (i, 0)),
            pl.BlockSpec((bn, H2 * C2), lambda i: (i, 0)),
            pl.BlockSpec((NC, bn, 80), lambda i: (0, i, 0)),
            pl.BlockSpec((NC, bn, 32), lambda i: (0, i, 0)),
            pl.BlockSpec(We2.shape, lambda i: (0, 0)),
            pl.BlockSpec((1, H2 * C2), lambda i: (0, 0)),
            pl.BlockSpec((1, H2 * C2), lambda i: (0, 0)),
            pl.BlockSpec((1, H2 * C2), lambda i: (0, 0)),
            pl.BlockSpec((1, H2 * C2), lambda i: (0, 0)),
        ],
        out_specs=pl.BlockSpec((bn, H2 * C2), lambda i: (i, 0)),
        out_shape=jax.ShapeDtypeStruct((n_pad, H2 * C2), f32),
    )(xl2[:n_pad], xr2[:n_pad], num2, asum, We2,
      row2d(att2.reshape(-1)), row2d(bias2), row2d(g2), row2d(b2))

    return out[:n_nodes]


# speedup vs baseline: 33.8694x; 2.3962x over previous
import functools

import jax
import jax.numpy as jnp
from jax import lax
from jax.experimental import pallas as pl
from jax.experimental.pallas import tpu as pltpu
from jax.experimental.pallas import tpu_sc as plsc

H1, C1 = 4, 16
H2, C2 = 1, 64
NEG = 0.2
NC, NS, LANES = 2, 16, 16
NW = NC * NS
B_EDGE = 128
IDX_CHUNK = 8


def _mm_t(a, w):
    return lax.dot_general(a, w, (((1,), (1,)), ((), ())),
                           preferred_element_type=jnp.float32)


def _node_proj_body(x_ref, wl_ref, bl_ref, wr_ref, br_ref, xl_ref, xr_ref):
    xb = x_ref[...]
    xl_ref[...] = _mm_t(xb, wl_ref[...]) + bl_ref[...]
    xr_ref[...] = _mm_t(xb, wr_ref[...]) + br_ref[...]


def _edge_proj_body(ea_ref, we1_ref, we2_ref, ep1_ref, ep2_ref):
    ea = ea_ref[...]
    ep1_ref[...] = _mm_t(ea, we1_ref[...])
    ep2_ref[...] = _mm_t(ea, we2_ref[...])


def _lrelu(m):
    return jnp.maximum(m, NEG * m)


def _elu(h):
    return jnp.where(h > 0.0, h, jnp.exp(jnp.minimum(h, 0.0)) - 1.0)


def _ln_act(h, g_ref, b_ref):
    mu = jnp.mean(h, axis=-1, keepdims=True)
    var = jnp.mean((h - mu) * (h - mu), axis=-1, keepdims=True)
    h = (h - mu) * lax.rsqrt(var + 1e-5) * g_ref[...] + b_ref[...]
    return _elu(h)


def _epi1_body(xl_ref, xr_ref, num_ref, asum_ref, we1_ref, att_ref, p_ref,
               bias_ref, g_ref, b_ref, wl2_ref, bl2_ref, wr2_ref, br2_ref,
               xl2_ref, xr2_ref):
    asum = asum_ref[0] + asum_ref[1]
    attr = asum[:, 0:16]
    deg = asum[:, 16:17]
    loop_attr = attr / jnp.maximum(deg, 1.0)
    epl = _mm_t(loop_attr, we1_ref[...])
    xl = xl_ref[...]
    xr = xr_ref[...]
    m = _lrelu(xl + xr + epl)
    p = p_ref[...]
    a_self = lax.dot_general(m * att_ref[...], p, (((1,), (0,)), ((), ())),
                             preferred_element_type=jnp.float32)
    w_self = jnp.exp(a_self)
    wfull = lax.dot_general(w_self, p, (((1,), (1,)), ((), ())),
                            preferred_element_type=jnp.float32)
    nump = num_ref[0] + num_ref[1]
    num = nump[:, 0:64] + xl * wfull
    den_h = nump[:, 64:64 + H1] + w_self
    den = lax.dot_general(den_h, p, (((1,), (1,)), ((), ())),
                          preferred_element_type=jnp.float32)
    out = num / den + bias_ref[...]
    h = _ln_act(out, g_ref, b_ref)
    xl2_ref[...] = _mm_t(h, wl2_ref[...]) + bl2_ref[...]
    xr2_ref[...] = _mm_t(h, wr2_ref[...]) + br2_ref[...]


def _epi2_body(xl_ref, xr_ref, num_ref, asum_ref, we2_ref, att_ref,
               bias_ref, g_ref, b_ref, out_ref):
    asum = asum_ref[0] + asum_ref[1]
    attr = asum[:, 0:16]
    deg = asum[:, 16:17]
    loop_attr = attr / jnp.maximum(deg, 1.0)
    epl = _mm_t(loop_attr, we2_ref[...])
    xl = xl_ref[...]
    xr = xr_ref[...]
    m = _lrelu(xl + xr + epl)
    a_self = jnp.sum(m * att_ref[...], axis=-1, keepdims=True)
    w_self = jnp.exp(a_self)
    nump = num_ref[0] + num_ref[1]
    num = nump[:, 0:64] + xl * w_self
    den = nump[:, 64:65] + w_self
    out = num / den + bias_ref[...]
    out_ref[...] = _ln_act(out, g_ref, b_ref)


def _zero_stage(stage, rows, cols):
    z = jnp.zeros((LANES,), jnp.float32)

    def zr(r, _):
        for k in range(cols // LANES):
            stage[r, pl.ds(k * LANES, LANES)] = z
        return 0

    lax.fori_loop(0, rows, zr, 0)


def _zero_shared_rows(stage, accum, base, rows):
    off = 0
    while off < rows:
        c = min(B_EDGE, rows - off)
        pltpu.sync_copy(stage.at[pl.ds(0, c)], accum.at[pl.ds(base + off, c)])
        off += c


def _export_shared_rows(accum, out, cid, base, rows):
    off = 0
    while off < rows:
        c = min(B_EDGE, rows - off)
        pltpu.sync_copy(accum.at[pl.ds(base + off, c)],
                        out.at[cid, pl.ds(base + off, c)])
        off += c


def _make_attr_sum_kernel(nacc, e_pad):
    epw = e_pad // NW
    nb = epw // B_EDGE
    rows_pt = nacc // NS
    mesh = plsc.VectorSubcoreMesh(core_axis_name="c", subcore_axis_name="s",
                                  num_cores=NC, num_subcores=NS)

    @functools.partial(
        pl.kernel,
        out_type=jax.ShapeDtypeStruct((NC, nacc, 32), jnp.float32),
        mesh=mesh,
        compiler_params=pltpu.CompilerParams(needs_layout_passes=False, use_tc_tiling_on_sc=False),
        scratch_types=[
            pltpu.VMEM_SHARED((nacc, 32), jnp.float32),
            pltpu.VMEM((B_EDGE,), jnp.int32),
            pltpu.VMEM((B_EDGE, 16), jnp.float32),
            pltpu.VMEM((B_EDGE, 32), jnp.float32),
        ],
    )
    def body(ea_hbm, dst_hbm, out_hbm, accum, idx_d, ea_v, stage):
        cid = lax.axis_index("c")
        sid = lax.axis_index("s")
        wid = cid * NS + sid
        _zero_stage(stage, B_EDGE, 32)
        _zero_shared_rows(stage, accum, sid * rows_pt, rows_pt)
        ones = jnp.ones((LANES,), jnp.float32)
        c16 = jnp.full((LANES,), 16, jnp.int32)

        def gset(j, _):
            ev = lax.iota(jnp.int32, LANES) + j * LANES
            plsc.store_scatter(stage, [ev, c16], ones)
            return 0

        lax.fori_loop(0, B_EDGE // LANES, gset, 0)
        plsc.subcore_barrier()

        def batch(b, _):
            e0 = wid * epw + b * B_EDGE
            pltpu.sync_copy(dst_hbm.at[pl.ds(e0, B_EDGE)], idx_d)
            pltpu.sync_copy(ea_hbm.at[pl.ds(e0, B_EDGE)], ea_v)

            def edge(e, _):
                stage[e, pl.ds(0, LANES)] = ea_v[e, pl.ds(0, LANES)]
                return 0

            lax.fori_loop(0, B_EDGE, edge, 0, unroll=4)
            pltpu.sync_copy(stage, accum.at[idx_d], add=True)
            return 0

        lax.fori_loop(0, nb, batch, 0)
        plsc.subcore_barrier()
        _export_shared_rows(accum, out_hbm, cid, sid * rows_pt, rows_pt)

    return body


def _make_edge_kernel(h_heads, c_dim, nacc, e_pad):
    epw = e_pad // NW
    nb = epw // B_EDGE
    assert nb % 2 == 0 and nb % IDX_CHUNK == 0
    rows_pt = nacc // NS
    mesh = plsc.VectorSubcoreMesh(core_axis_name="c", subcore_axis_name="s",
                                  num_cores=NC, num_subcores=NS)

    @functools.partial(
        pl.kernel,
        out_type=jax.ShapeDtypeStruct((NC, nacc, 80), jnp.float32),
        mesh=mesh,
        compiler_params=pltpu.CompilerParams(needs_layout_passes=False, use_tc_tiling_on_sc=False),
        scratch_types=[
            pltpu.VMEM_SHARED((nacc, 80), jnp.float32),
            pltpu.VMEM((IDX_CHUNK, B_EDGE), jnp.int32),
            pltpu.VMEM((IDX_CHUNK, B_EDGE), jnp.int32),
            pltpu.VMEM((IDX_CHUNK, B_EDGE), jnp.int32),
            pltpu.VMEM((IDX_CHUNK, B_EDGE), jnp.int32),
            pltpu.VMEM((B_EDGE, 64), jnp.float32),
            pltpu.VMEM((B_EDGE, 64), jnp.float32),
            pltpu.VMEM((B_EDGE, 64), jnp.float32),
            pltpu.VMEM((B_EDGE, 64), jnp.float32),
            pltpu.VMEM((B_EDGE, 64), jnp.float32),
            pltpu.VMEM((B_EDGE, 64), jnp.float32),
            pltpu.VMEM((B_EDGE, 80), jnp.float32),
            pltpu.VMEM((B_EDGE, 80), jnp.float32),
            pltpu.VMEM((80,), jnp.float32),
            pltpu.SemaphoreType.DMA,
            pltpu.SemaphoreType.DMA,
            pltpu.SemaphoreType.DMA,
            pltpu.SemaphoreType.DMA,
            pltpu.SemaphoreType.DMA,
            pltpu.SemaphoreType.DMA,
            pltpu.SemaphoreType.DMA,
            pltpu.SemaphoreType.DMA,
        ],
    )
    def body(xl_hbm, xr_hbm, ep_hbm, src_hbm, dst_hbm, att_hbm, out_hbm,
             accum, ixs0, ixs1, ixd0, ixd1,
             xl_a, xl_b, xr_a, xr_b, ep_a, ep_b, st_a, st_b, att_v,
             gx0, gx1, gr0, gr1, ge0, ge1, ss0, ss1):
        cid = lax.axis_index("c")
        sid = lax.axis_index("s")
        wid = cid * NS + sid
        xl_s = (xl_a, xl_b)
        xr_s = (xr_a, xr_b)
        ep_s = (ep_a, ep_b)
        st_s = (st_a, st_b)
        ixs = (ixs0, ixs1)
        ixd = (ixd0, ixd1)
        gx = (gx0, gx1)
        gr = (gr0, gr1)
        ge = (ge0, ge1)
        ss = (ss0, ss1)
        _zero_stage(st_a, B_EDGE, 80)
        _zero_stage(st_b, B_EDGE, 80)
        _zero_shared_rows(st_a, accum, sid * rows_pt, rows_pt)
        pltpu.sync_copy(att_hbm, att_v.at[pl.ds(16, 64)])
        plsc.subcore_barrier()

        def fetch_chunk(c, par):
            pltpu.sync_copy(src_hbm.at[wid, pl.ds(c * IDX_CHUNK, IDX_CHUNK)],
                            ixs[par])
            pltpu.sync_copy(dst_hbm.at[wid, pl.ds(c * IDX_CHUNK, IDX_CHUNK)],
                            ixd[par])

        def issue_gathers(bi, slot, par):
            r = bi % IDX_CHUNK
            pltpu.async_copy(xl_hbm.at[ixs[par].at[r]], xl_s[slot], gx[slot])
            pltpu.async_copy(xr_hbm.at[ixd[par].at[r]], xr_s[slot], gr[slot])

        def compute(slot):
            xlv, xrv, epv, stage = xl_s[slot], xr_s[slot], ep_s[slot], st_s[slot]
            nblk = (h_heads * c_dim) // LANES
            atts = [att_v[pl.ds(16 + LANES * k, LANES)] for k in range(nblk)]
            lane = lax.iota(jnp.int32, LANES)
            ninf = jnp.full((LANES,), -1e30, jnp.float32)

            def edge(e, _):
                xs = [xlv[e, pl.ds(LANES * k, LANES)] for k in range(nblk)]
                us = []
                for k in range(nblk):
                    u = xs[k] + xrv[e, pl.ds(LANES * k, LANES)] \
                        + epv[e, pl.ds(LANES * k, LANES)]
                    us.append(_lrelu(u) * atts[k])
                if h_heads == nblk:
                    apk = ninf
                    for k in range(nblk):
                        ak = jnp.sum(us[k], axis=0)
                        apk = jnp.where(lane == k,
                                        jnp.broadcast_to(ak, (LANES,)), apk)
                else:
                    t = us[0]
                    for k in range(1, nblk):
                        t = t + us[k]
                    a = jnp.sum(t, axis=0)
                    apk = jnp.where(lane == 0,
                                    jnp.broadcast_to(a, (LANES,)), ninf)
                wv = jnp.exp(apk)
                for k in range(nblk):
                    h = k if h_heads == nblk else 0
                    wk = jnp.broadcast_to(wv[h], (LANES,))
                    stage[e, pl.ds(LANES * k, LANES)] = xs[k] * wk
                stage[e, pl.ds(64, LANES)] = wv
                return 0

            lax.fori_loop(0, B_EDGE, edge, 0, unroll=2)

        def wait_gathers(slot):
            pltpu.make_async_copy(xl_hbm.at[ixs[0].at[0]], xl_s[slot],
                                  gx[slot]).wait()
            pltpu.make_async_copy(xr_hbm.at[ixd[0].at[0]], xr_s[slot],
                                  gr[slot]).wait()
            pltpu.make_async_copy(ep_hbm.at[pl.ds(0, B_EDGE)], ep_s[slot],
                                  ge[slot]).wait()

        def wait_scatter(slot):
            pltpu.make_async_copy(st_s[slot], accum.at[ixd[0].at[0]],
                                  ss[slot]).wait()

        def phase(i, slot):
            other = 1 - slot
            @pl.when(i >= 2)
            def _():
                wait_scatter(slot)

            wait_gathers(slot)
            nxt = i + 1

            @pl.when(nxt < nb)
            def _():
                cpar = (nxt // IDX_CHUNK) % 2
                refresh = (nxt % IDX_CHUNK) == 0

                @pl.when(jnp.logical_and(refresh, cpar == 0))
                def _():
                    fetch_chunk(nxt // IDX_CHUNK, 0)

                @pl.when(jnp.logical_and(refresh, cpar == 1))
                def _():
                    fetch_chunk(nxt // IDX_CHUNK, 1)

                @pl.when(cpar == 0)
                def _():
                    issue_gathers(nxt, other, 0)

                @pl.when(cpar == 1)
                def _():
                    issue_gathers(nxt, other, 1)

                pltpu.async_copy(
                    ep_hbm.at[pl.ds(wid * epw + nxt * B_EDGE, B_EDGE)],
                    ep_s[other], ge[other])

            compute(slot)
            rcur = i % IDX_CHUNK
            ccur = (i // IDX_CHUNK) % 2
            @pl.when(ccur == 0)
            def _():
                pltpu.async_copy(st_s[slot], accum.at[ixd[0].at[rcur]],
                                 ss[slot], add=True)

            @pl.when(ccur == 1)
            def _():
                pltpu.async_copy(st_s[slot], accum.at[ixd[1].at[rcur]],
                                 ss[slot], add=True)

        fetch_chunk(0, 0)
        issue_gathers(0, 0, 0)
        pltpu.async_copy(ep_hbm.at[pl.ds(wid * epw, B_EDGE)], ep_a, ge0)

        def pair(k, _):
            phase(2 * k, 0)
            phase(2 * k + 1, 1)
            return 0

        lax.fori_loop(0, nb // 2, pair, 0)
        wait_scatter(0)
        wait_scatter(1)
        plsc.subcore_barrier()
        _export_shared_rows(accum, out_hbm, cid, sid * rows_pt, rows_pt)

    return body


def kernel(x, edge_index, edge_attr, Wl1, bl1, Wr1, br1, We1, att1, bias1,
           g1, b1, Wl2, bl2, Wr2, br2, We2, att2, bias2, g2, b2):
    n_nodes, d_in = x.shape
    e_edges = edge_index.shape[1]
    f32 = jnp.float32

    src = edge_index[0]
    dst = edge_index[1]

    chunk = NW * B_EDGE * IDX_CHUNK
    e_pad = -(-e_edges // chunk) * chunk
    nacc = -(-(n_nodes + 1) // (NS * 8)) * (NS * 8)
    if e_pad != e_edges:
        pad = e_pad - e_edges
        src = jnp.concatenate([src, jnp.zeros((pad,), src.dtype)])
        dst = jnp.concatenate([dst, jnp.full((pad,), n_nodes, dst.dtype)])
        edge_attr = jnp.concatenate(
            [edge_attr, jnp.zeros((pad, edge_attr.shape[1]), edge_attr.dtype)])

    bn = 1000
    n_pad = -(-n_nodes // bn) * bn
    if n_pad != n_nodes:
        x = jnp.concatenate([x, jnp.zeros((n_pad - n_nodes, d_in), f32)])
    n_blocks = n_pad // bn

    row2d = lambda a: a.reshape(1, -1).astype(f32)

    xl1, xr1 = pl.pallas_call(
        _node_proj_body,
        grid=(n_blocks,),
        in_specs=[
            pl.BlockSpec((bn, d_in), lambda i: (i, 0)),
            pl.BlockSpec(Wl1.shape, lambda i: (0, 0)),
            pl.BlockSpec((1, H1 * C1), lambda i: (0, 0)),
            pl.BlockSpec(Wr1.shape, lambda i: (0, 0)),
            pl.BlockSpec((1, H1 * C1), lambda i: (0, 0)),
        ],
        out_specs=[
            pl.BlockSpec((bn, H1 * C1), lambda i: (i, 0)),
            pl.BlockSpec((bn, H1 * C1), lambda i: (i, 0)),
        ],
        out_shape=[
            jax.ShapeDtypeStruct((n_pad, H1 * C1), f32),
            jax.ShapeDtypeStruct((n_pad, H1 * C1), f32),
        ],
    )(x, Wl1, row2d(bl1), Wr1, row2d(br1))

    be = 6400
    eb = e_pad // be
    ep1, ep2 = pl.pallas_call(
        _edge_proj_body,
        grid=(eb,),
        in_specs=[
            pl.BlockSpec((be, edge_attr.shape[1]), lambda i: (i, 0)),
            pl.BlockSpec(We1.shape, lambda i: (0, 0)),
            pl.BlockSpec(We2.shape, lambda i: (0, 0)),
        ],
        out_specs=[
            pl.BlockSpec((be, H1 * C1), lambda i: (i, 0)),
            pl.BlockSpec((be, H2 * C2), lambda i: (i, 0)),
        ],
        out_shape=[
            jax.ShapeDtypeStruct((e_pad, H1 * C1), f32),
            jax.ShapeDtypeStruct((e_pad, H2 * C2), f32),
        ],
    )(edge_attr.astype(f32), We1, We2)

    asum = _make_attr_sum_kernel(nacc, e_pad)(edge_attr.astype(f32), dst)

    if e_pad == e_edges or n_pad > n_nodes:
        xl1g, xr1g = xl1, xr1
    else:
        zrow = jnp.zeros((1, H1 * C1), f32)
        xl1g = jnp.concatenate([xl1, zrow])
        xr1g = jnp.concatenate([xr1, zrow])
    nb_w = e_pad // NW // B_EDGE
    src3 = src.reshape(NW, nb_w, B_EDGE)
    dst3 = dst.reshape(NW, nb_w, B_EDGE)
    num1 = _make_edge_kernel(H1, C1, nacc, e_pad)(
        xl1g, xr1g, ep1, src3, dst3, att1.reshape(-1).astype(f32))

    pool = jnp.repeat(jnp.eye(H1, dtype=f32), C1, axis=0)
    xl2, xr2 = pl.pallas_call(
        _epi1_body,
        grid=(n_nodes // bn if n_nodes % bn == 0 else n_blocks,),
        in_specs=[
            pl.BlockSpec((bn, H1 * C1), lambda i: (i, 0)),
            pl.BlockSpec((bn, H1 * C1), lambda i: (i, 0)),
            pl.BlockSpec((NC, bn, 80), lambda i: (0, i, 0)),
            pl.BlockSpec((NC, bn, 32), lambda i: (0, i, 0)),
            pl.BlockSpec(We1.shape, lambda i: (0, 0)),
            pl.BlockSpec((1, H1 * C1), lambda i: (0, 0)),
            pl.BlockSpec((H1 * C1, H1), lambda i: (0, 0)),
            pl.BlockSpec((1, H1 * C1), lambda i: (0, 0)),
            pl.BlockSpec((1, H1 * C1), lambda i: (0, 0)),
            pl.BlockSpec((1, H1 * C1), lambda i: (0, 0)),
            pl.BlockSpec(Wl2.shape, lambda i: (0, 0)),
            pl.BlockSpec((1, H2 * C2), lambda i: (0, 0)),
            pl.BlockSpec(Wr2.shape, lambda i: (0, 0)),
            pl.BlockSpec((1, H2 * C2), lambda i: (0, 0)),
        ],
        out_specs=[
            pl.BlockSpec((bn, H2 * C2), lambda i: (i, 0)),
            pl.BlockSpec((bn, H2 * C2), lambda i: (i, 0)),
        ],
        out_shape=[
            jax.ShapeDtypeStruct((n_pad, H2 * C2), f32),
            jax.ShapeDtypeStruct((n_pad, H2 * C2), f32),
        ],
    )(xl1[:n_pad], xr1[:n_pad], num1, asum, We1,
      row2d(att1.reshape(-1)), pool, row2d(bias1), row2d(g1), row2d(b1),
      Wl2, row2d(bl2), Wr2, row2d(br2))

    if e_pad == e_edges or n_pad > n_nodes:
        xl2g, xr2g = xl2, xr2
    else:
        zrow = jnp.zeros((1, H2 * C2), f32)
        xl2g = jnp.concatenate([xl2, zrow])
        xr2g = jnp.concatenate([xr2, zrow])
    num2 = _make_edge_kernel(H2, C2, nacc, e_pad)(
        xl2g, xr2g, ep2, src3, dst3, att2.reshape(-1).astype(f32))

    out = pl.pallas_call(
        _epi2_body,
        grid=(n_blocks,),
        in_specs=[
            pl.BlockSpec((bn, H2 * C2), lambda i: (i, 0)),
            pl.BlockSpec((bn, H2 * C2), lambda i: (i, 0)),
            pl.BlockSpec((NC, bn, 80), lambda i: (0, i, 0)),
            pl.BlockSpec((NC, bn, 32), lambda i: (0, i, 0)),
            pl.BlockSpec(We2.shape, lambda i: (0, 0)),
            pl.BlockSpec((1, H2 * C2), lambda i: (0, 0)),
            pl.BlockSpec((1, H2 * C2), lambda i: (0, 0)),
            pl.BlockSpec((1, H2 * C2), lambda i: (0, 0)),
            pl.BlockSpec((1, H2 * C2), lambda i: (0, 0)),
        ],
        out_specs=pl.BlockSpec((bn, H2 * C2), lambda i: (i, 0)),
        out_shape=jax.ShapeDtypeStruct((n_pad, H2 * C2), f32),
    )(xl2[:n_pad], xr2[:n_pad], num2, asum, We2,
      row2d(att2.reshape(-1)), row2d(bias2), row2d(g2), row2d(b2))

    return out[:n_nodes]


# speedup vs baseline: 34.1548x; 1.0084x over previous
import functools

import jax
import jax.numpy as jnp
from jax import lax
from jax.experimental import pallas as pl
from jax.experimental.pallas import tpu as pltpu
from jax.experimental.pallas import tpu_sc as plsc

H1, C1 = 4, 16
H2, C2 = 1, 64
NEG = 0.2
NC, NS, LANES = 2, 16, 16
NW = NC * NS
B_EDGE = 128
IDX_CHUNK = 8


def _mm_t(a, w):
    return lax.dot_general(a, w, (((1,), (1,)), ((), ())),
                           preferred_element_type=jnp.float32)


def _node_proj_body(x_ref, wl_ref, bl_ref, wr_ref, br_ref, xl_ref, xr_ref):
    xb = x_ref[...]
    xl_ref[...] = _mm_t(xb, wl_ref[...]) + bl_ref[...]
    xr_ref[...] = _mm_t(xb, wr_ref[...]) + br_ref[...]


def _edge_proj_body(ea_ref, we1_ref, we2_ref, ep1_ref, ep2_ref):
    ea = ea_ref[...]
    ep1_ref[...] = _mm_t(ea, we1_ref[...])
    ep2_ref[...] = _mm_t(ea, we2_ref[...])


def _lrelu(m):
    return jnp.maximum(m, NEG * m)


def _elu(h):
    return jnp.where(h > 0.0, h, jnp.exp(jnp.minimum(h, 0.0)) - 1.0)


def _ln_act(h, g_ref, b_ref):
    mu = jnp.mean(h, axis=-1, keepdims=True)
    var = jnp.mean((h - mu) * (h - mu), axis=-1, keepdims=True)
    h = (h - mu) * lax.rsqrt(var + 1e-5) * g_ref[...] + b_ref[...]
    return _elu(h)


def _epi1_body(xl_ref, xr_ref, num_ref, asum_ref, we1_ref, att_ref, p_ref,
               bias_ref, g_ref, b_ref, wl2_ref, bl2_ref, wr2_ref, br2_ref,
               xl2_ref, xr2_ref):
    asum = asum_ref[0] + asum_ref[1]
    attr = asum[:, 0:16]
    deg = asum[:, 16:17]
    loop_attr = attr / jnp.maximum(deg, 1.0)
    epl = _mm_t(loop_attr, we1_ref[...])
    xl = xl_ref[...]
    xr = xr_ref[...]
    m = _lrelu(xl + xr + epl)
    p = p_ref[...]
    a_self = lax.dot_general(m * att_ref[...], p, (((1,), (0,)), ((), ())),
                             preferred_element_type=jnp.float32)
    w_self = jnp.exp(a_self)
    wfull = lax.dot_general(w_self, p, (((1,), (1,)), ((), ())),
                            preferred_element_type=jnp.float32)
    nump = num_ref[0] + num_ref[1]
    num = nump[:, 0:64] + xl * wfull
    den_h = nump[:, 64:64 + H1] + w_self
    den = lax.dot_general(den_h, p, (((1,), (1,)), ((), ())),
                          preferred_element_type=jnp.float32)
    out = num / den + bias_ref[...]
    h = _ln_act(out, g_ref, b_ref)
    xl2_ref[...] = _mm_t(h, wl2_ref[...]) + bl2_ref[...]
    xr2_ref[...] = _mm_t(h, wr2_ref[...]) + br2_ref[...]


def _epi2_body(xl_ref, xr_ref, num_ref, asum_ref, we2_ref, att_ref,
               bias_ref, g_ref, b_ref, out_ref):
    asum = asum_ref[0] + asum_ref[1]
    attr = asum[:, 0:16]
    deg = asum[:, 16:17]
    loop_attr = attr / jnp.maximum(deg, 1.0)
    epl = _mm_t(loop_attr, we2_ref[...])
    xl = xl_ref[...]
    xr = xr_ref[...]
    m = _lrelu(xl + xr + epl)
    a_self = jnp.sum(m * att_ref[...], axis=-1, keepdims=True)
    w_self = jnp.exp(a_self)
    nump = num_ref[0] + num_ref[1]
    num = nump[:, 0:64] + xl * w_self
    den = nump[:, 64:65] + w_self
    out = num / den + bias_ref[...]
    out_ref[...] = _ln_act(out, g_ref, b_ref)


def _zero_stage(stage, rows, cols):
    z = jnp.zeros((LANES,), jnp.float32)

    def zr(r, _):
        for k in range(cols // LANES):
            stage[r, pl.ds(k * LANES, LANES)] = z
        return 0

    lax.fori_loop(0, rows, zr, 0)


def _zero_shared_rows(stage, accum, base, rows):
    off = 0
    while off < rows:
        c = min(B_EDGE, rows - off)
        pltpu.sync_copy(stage.at[pl.ds(0, c)], accum.at[pl.ds(base + off, c)])
        off += c


def _export_shared_rows(accum, out, cid, base, rows):
    off = 0
    while off < rows:
        c = min(B_EDGE, rows - off)
        pltpu.sync_copy(accum.at[pl.ds(base + off, c)],
                        out.at[cid, pl.ds(base + off, c)])
        off += c


def _make_attr_sum_kernel(nacc, e_pad):
    epw = e_pad // NW
    nb = epw // B_EDGE
    rows_pt = nacc // NS
    mesh = plsc.VectorSubcoreMesh(core_axis_name="c", subcore_axis_name="s",
                                  num_cores=NC, num_subcores=NS)

    @functools.partial(
        pl.kernel,
        out_type=jax.ShapeDtypeStruct((NC, nacc, 32), jnp.float32),
        mesh=mesh,
        compiler_params=pltpu.CompilerParams(needs_layout_passes=False, use_tc_tiling_on_sc=False),
        scratch_types=[
            pltpu.VMEM_SHARED((nacc, 32), jnp.float32),
            pltpu.VMEM((B_EDGE,), jnp.int32),
            pltpu.VMEM((B_EDGE, 16), jnp.float32),
            pltpu.VMEM((B_EDGE, 32), jnp.float32),
        ],
    )
    def body(ea_hbm, dst_hbm, out_hbm, accum, idx_d, ea_v, stage):
        cid = lax.axis_index("c")
        sid = lax.axis_index("s")
        wid = cid * NS + sid
        _zero_stage(stage, B_EDGE, 32)
        _zero_shared_rows(stage, accum, sid * rows_pt, rows_pt)
        ones = jnp.ones((LANES,), jnp.float32)
        c16 = jnp.full((LANES,), 16, jnp.int32)

        def gset(j, _):
            ev = lax.iota(jnp.int32, LANES) + j * LANES
            plsc.store_scatter(stage, [ev, c16], ones)
            return 0

        lax.fori_loop(0, B_EDGE // LANES, gset, 0)
        plsc.subcore_barrier()

        def batch(b, _):
            e0 = wid * epw + b * B_EDGE
            pltpu.sync_copy(dst_hbm.at[pl.ds(e0, B_EDGE)], idx_d)
            pltpu.sync_copy(ea_hbm.at[pl.ds(e0, B_EDGE)], ea_v)

            def edge(e, _):
                stage[e, pl.ds(0, LANES)] = ea_v[e, pl.ds(0, LANES)]
                return 0

            lax.fori_loop(0, B_EDGE, edge, 0, unroll=4)
            pltpu.sync_copy(stage, accum.at[idx_d], add=True)
            return 0

        lax.fori_loop(0, nb, batch, 0)
        plsc.subcore_barrier()
        _export_shared_rows(accum, out_hbm, cid, sid * rows_pt, rows_pt)

    return body


def _make_edge_kernel(h_heads, c_dim, nacc, e_pad):
    epw = e_pad // NW
    nb = epw // B_EDGE
    assert nb % 2 == 0 and nb % IDX_CHUNK == 0
    rows_pt = nacc // NS
    mesh = plsc.VectorSubcoreMesh(core_axis_name="c", subcore_axis_name="s",
                                  num_cores=NC, num_subcores=NS)

    @functools.partial(
        pl.kernel,
        out_type=jax.ShapeDtypeStruct((NC, nacc, 80), jnp.float32),
        mesh=mesh,
        compiler_params=pltpu.CompilerParams(needs_layout_passes=False, use_tc_tiling_on_sc=False),
        scratch_types=[
            pltpu.VMEM_SHARED((nacc, 80), jnp.float32),
            pltpu.VMEM((IDX_CHUNK, B_EDGE), jnp.int32),
            pltpu.VMEM((IDX_CHUNK, B_EDGE), jnp.int32),
            pltpu.VMEM((IDX_CHUNK, B_EDGE), jnp.int32),
            pltpu.VMEM((IDX_CHUNK, B_EDGE), jnp.int32),
            pltpu.VMEM((B_EDGE, 64), jnp.float32),
            pltpu.VMEM((B_EDGE, 64), jnp.float32),
            pltpu.VMEM((B_EDGE, 64), jnp.float32),
            pltpu.VMEM((B_EDGE, 64), jnp.float32),
            pltpu.VMEM((B_EDGE, 64), jnp.float32),
            pltpu.VMEM((B_EDGE, 64), jnp.float32),
            pltpu.VMEM((B_EDGE, 80), jnp.float32),
            pltpu.VMEM((B_EDGE, 80), jnp.float32),
            pltpu.VMEM((80,), jnp.float32),
            pltpu.SemaphoreType.DMA,
            pltpu.SemaphoreType.DMA,
            pltpu.SemaphoreType.DMA,
            pltpu.SemaphoreType.DMA,
            pltpu.SemaphoreType.DMA,
            pltpu.SemaphoreType.DMA,
            pltpu.SemaphoreType.DMA,
            pltpu.SemaphoreType.DMA,
        ],
    )
    def body(xl_hbm, xr_hbm, ep_hbm, src_hbm, dst_hbm, att_hbm, out_hbm,
             accum, ixs0, ixs1, ixd0, ixd1,
             xl_a, xl_b, xr_a, xr_b, ep_a, ep_b, st_a, st_b, att_v,
             gx0, gx1, gr0, gr1, ge0, ge1, ss0, ss1):
        cid = lax.axis_index("c")
        sid = lax.axis_index("s")
        wid = cid * NS + sid
        xl_s = (xl_a, xl_b)
        xr_s = (xr_a, xr_b)
        ep_s = (ep_a, ep_b)
        st_s = (st_a, st_b)
        ixs = (ixs0, ixs1)
        ixd = (ixd0, ixd1)
        gx = (gx0, gx1)
        gr = (gr0, gr1)
        ge = (ge0, ge1)
        ss = (ss0, ss1)
        _zero_stage(st_a, B_EDGE, 80)
        _zero_stage(st_b, B_EDGE, 80)
        _zero_shared_rows(st_a, accum, sid * rows_pt, rows_pt)
        pltpu.sync_copy(att_hbm, att_v.at[pl.ds(16, 64)])
        plsc.subcore_barrier()

        def fetch_chunk(c, par):
            pltpu.sync_copy(src_hbm.at[wid, pl.ds(c * IDX_CHUNK, IDX_CHUNK)],
                            ixs[par])
            pltpu.sync_copy(dst_hbm.at[wid, pl.ds(c * IDX_CHUNK, IDX_CHUNK)],
                            ixd[par])

        def issue_gathers(bi, slot, par):
            r = bi % IDX_CHUNK
            pltpu.async_copy(xl_hbm.at[ixs[par].at[r]], xl_s[slot], gx[slot])
            pltpu.async_copy(xr_hbm.at[ixd[par].at[r]], xr_s[slot], gr[slot])

        def compute(slot):
            xlv, xrv, epv, stage = xl_s[slot], xr_s[slot], ep_s[slot], st_s[slot]
            nblk = (h_heads * c_dim) // LANES
            atts = [att_v[pl.ds(16 + LANES * k, LANES)] for k in range(nblk)]
            lane = lax.iota(jnp.int32, LANES)
            ninf = jnp.full((LANES,), -1e30, jnp.float32)

            def edge(e, _):
                xs = [xlv[e, pl.ds(LANES * k, LANES)] for k in range(nblk)]
                us = []
                for k in range(nblk):
                    u = xs[k] + xrv[e, pl.ds(LANES * k, LANES)] \
                        + epv[e, pl.ds(LANES * k, LANES)]
                    us.append(_lrelu(u) * atts[k])
                if h_heads == nblk:
                    apk = ninf
                    for k in range(nblk):
                        ak = jnp.sum(us[k], axis=0)
                        apk = jnp.where(lane == k,
                                        jnp.broadcast_to(ak, (LANES,)), apk)
                else:
                    t = us[0]
                    for k in range(1, nblk):
                        t = t + us[k]
                    a = jnp.sum(t, axis=0)
                    apk = jnp.where(lane == 0,
                                    jnp.broadcast_to(a, (LANES,)), ninf)
                wv = jnp.exp(apk)
                for k in range(nblk):
                    h = k if h_heads == nblk else 0
                    wk = jnp.broadcast_to(wv[h], (LANES,))
                    stage[e, pl.ds(LANES * k, LANES)] = xs[k] * wk
                stage[e, pl.ds(64, LANES)] = wv
                return 0

            lax.fori_loop(0, B_EDGE, edge, 0, unroll=4)

        def wait_gathers(slot):
            pltpu.make_async_copy(xl_hbm.at[ixs[0].at[0]], xl_s[slot],
                                  gx[slot]).wait()
            pltpu.make_async_copy(xr_hbm.at[ixd[0].at[0]], xr_s[slot],
                                  gr[slot]).wait()
            pltpu.make_async_copy(ep_hbm.at[pl.ds(0, B_EDGE)], ep_s[slot],
                                  ge[slot]).wait()

        def wait_scatter(slot):
            pltpu.make_async_copy(st_s[slot], accum.at[ixd[0].at[0]],
                                  ss[slot]).wait()

        def phase(i, slot):
            other = 1 - slot
            @pl.when(i >= 2)
            def _():
                wait_scatter(slot)

            wait_gathers(slot)
            nxt = i + 1

            @pl.when(nxt < nb)
            def _():
                cpar = (nxt // IDX_CHUNK) % 2
                refresh = (nxt % IDX_CHUNK) == 0

                @pl.when(jnp.logical_and(refresh, cpar == 0))
                def _():
                    fetch_chunk(nxt // IDX_CHUNK, 0)

                @pl.when(jnp.logical_and(refresh, cpar == 1))
                def _():
                    fetch_chunk(nxt // IDX_CHUNK, 1)

                @pl.when(cpar == 0)
                def _():
                    issue_gathers(nxt, other, 0)

                @pl.when(cpar == 1)
                def _():
                    issue_gathers(nxt, other, 1)

                pltpu.async_copy(
                    ep_hbm.at[pl.ds(wid * epw + nxt * B_EDGE, B_EDGE)],
                    ep_s[other], ge[other])

            compute(slot)
            rcur = i % IDX_CHUNK
            ccur = (i // IDX_CHUNK) % 2
            @pl.when(ccur == 0)
            def _():
                pltpu.async_copy(st_s[slot], accum.at[ixd[0].at[rcur]],
                                 ss[slot], add=True)

            @pl.when(ccur == 1)
            def _():
                pltpu.async_copy(st_s[slot], accum.at[ixd[1].at[rcur]],
                                 ss[slot], add=True)

        fetch_chunk(0, 0)
        issue_gathers(0, 0, 0)
        pltpu.async_copy(ep_hbm.at[pl.ds(wid * epw, B_EDGE)], ep_a, ge0)

        def pair(k, _):
            phase(2 * k, 0)
            phase(2 * k + 1, 1)
            return 0

        lax.fori_loop(0, nb // 2, pair, 0)
        wait_scatter(0)
        wait_scatter(1)
        plsc.subcore_barrier()
        _export_shared_rows(accum, out_hbm, cid, sid * rows_pt, rows_pt)

    return body


def kernel(x, edge_index, edge_attr, Wl1, bl1, Wr1, br1, We1, att1, bias1,
           g1, b1, Wl2, bl2, Wr2, br2, We2, att2, bias2, g2, b2):
    n_nodes, d_in = x.shape
    e_edges = edge_index.shape[1]
    f32 = jnp.float32

    src = edge_index[0]
    dst = edge_index[1]

    chunk = NW * B_EDGE * IDX_CHUNK
    e_pad = -(-e_edges // chunk) * chunk
    nacc = -(-(n_nodes + 1) // (NS * 8)) * (NS * 8)
    if e_pad != e_edges:
        pad = e_pad - e_edges
        src = jnp.concatenate([src, jnp.zeros((pad,), src.dtype)])
        dst = jnp.concatenate([dst, jnp.full((pad,), n_nodes, dst.dtype)])
        edge_attr = jnp.concatenate(
            [edge_attr, jnp.zeros((pad, edge_attr.shape[1]), edge_attr.dtype)])

    bn = 1000
    n_pad = -(-n_nodes // bn) * bn
    if n_pad != n_nodes:
        x = jnp.concatenate([x, jnp.zeros((n_pad - n_nodes, d_in), f32)])
    n_blocks = n_pad // bn

    row2d = lambda a: a.reshape(1, -1).astype(f32)

    xl1, xr1 = pl.pallas_call(
        _node_proj_body,
        grid=(n_blocks,),
        in_specs=[
            pl.BlockSpec((bn, d_in), lambda i: (i, 0)),
            pl.BlockSpec(Wl1.shape, lambda i: (0, 0)),
            pl.BlockSpec((1, H1 * C1), lambda i: (0, 0)),
            pl.BlockSpec(Wr1.shape, lambda i: (0, 0)),
            pl.BlockSpec((1, H1 * C1), lambda i: (0, 0)),
        ],
        out_specs=[
            pl.BlockSpec((bn, H1 * C1), lambda i: (i, 0)),
            pl.BlockSpec((bn, H1 * C1), lambda i: (i, 0)),
        ],
        out_shape=[
            jax.ShapeDtypeStruct((n_pad, H1 * C1), f32),
            jax.ShapeDtypeStruct((n_pad, H1 * C1), f32),
        ],
    )(x, Wl1, row2d(bl1), Wr1, row2d(br1))

    be = 6400
    eb = e_pad // be
    ep1, ep2 = pl.pallas_call(
        _edge_proj_body,
        grid=(eb,),
        in_specs=[
            pl.BlockSpec((be, edge_attr.shape[1]), lambda i: (i, 0)),
            pl.BlockSpec(We1.shape, lambda i: (0, 0)),
            pl.BlockSpec(We2.shape, lambda i: (0, 0)),
        ],
        out_specs=[
            pl.BlockSpec((be, H1 * C1), lambda i: (i, 0)),
            pl.BlockSpec((be, H2 * C2), lambda i: (i, 0)),
        ],
        out_shape=[
            jax.ShapeDtypeStruct((e_pad, H1 * C1), f32),
            jax.ShapeDtypeStruct((e_pad, H2 * C2), f32),
        ],
    )(edge_attr.astype(f32), We1, We2)

    asum = _make_attr_sum_kernel(nacc, e_pad)(edge_attr.astype(f32), dst)

    if e_pad == e_edges or n_pad > n_nodes:
        xl1g, xr1g = xl1, xr1
    else:
        zrow = jnp.zeros((1, H1 * C1), f32)
        xl1g = jnp.concatenate([xl1, zrow])
        xr1g = jnp.concatenate([xr1, zrow])
    nb_w = e_pad // NW // B_EDGE
    src3 = src.reshape(NW, nb_w, B_EDGE)
    dst3 = dst.reshape(NW, nb_w, B_EDGE)
    num1 = _make_edge_kernel(H1, C1, nacc, e_pad)(
        xl1g, xr1g, ep1, src3, dst3, att1.reshape(-1).astype(f32))

    pool = jnp.repeat(jnp.eye(H1, dtype=f32), C1, axis=0)
    xl2, xr2 = pl.pallas_call(
        _epi1_body,
        grid=(n_nodes // bn if n_nodes % bn == 0 else n_blocks,),
        in_specs=[
            pl.BlockSpec((bn, H1 * C1), lambda i: (i, 0)),
            pl.BlockSpec((bn, H1 * C1), lambda i: (i, 0)),
            pl.BlockSpec((NC, bn, 80), lambda i: (0, i, 0)),
            pl.BlockSpec((NC, bn, 32), lambda i: (0, i, 0)),
            pl.BlockSpec(We1.shape, lambda i: (0, 0)),
            pl.BlockSpec((1, H1 * C1), lambda i: (0, 0)),
            pl.BlockSpec((H1 * C1, H1), lambda i: (0, 0)),
            pl.BlockSpec((1, H1 * C1), lambda i: (0, 0)),
            pl.BlockSpec((1, H1 * C1), lambda i: (0, 0)),
            pl.BlockSpec((1, H1 * C1), lambda i: (0, 0)),
            pl.BlockSpec(Wl2.shape, lambda i: (0, 0)),
            pl.BlockSpec((1, H2 * C2), lambda i: (0, 0)),
            pl.BlockSpec(Wr2.shape, lambda i: (0, 0)),
            pl.BlockSpec((1, H2 * C2), lambda i: (0, 0)),
        ],
        out_specs=[
            pl.BlockSpec((bn, H2 * C2), lambda i: (i, 0)),
            pl.BlockSpec((bn, H2 * C2), lambda i: (i, 0)),
        ],
        out_shape=[
            jax.ShapeDtypeStruct((n_pad, H2 * C2), f32),
            jax.ShapeDtypeStruct((n_pad, H2 * C2), f32),
        ],
    )(xl1[:n_pad], xr1[:n_pad], num1, asum, We1,
      row2d(att1.reshape(-1)), pool, row2d(bias1), row2d(g1), row2d(b1),
      Wl2, row2d(bl2), Wr2, row2d(br2))

    if e_pad == e_edges or n_pad > n_nodes:
        xl2g, xr2g = xl2, xr2
    else:
        zrow = jnp.zeros((1, H2 * C2), f32)
        xl2g = jnp.concatenate([xl2, zrow])
        xr2g = jnp.concatenate([xr2, zrow])
    num2 = _make_edge_kernel(H2, C2, nacc, e_pad)(
        xl2g, xr2g, ep2, src3, dst3, att2.reshape(-1).astype(f32))

    out = pl.pallas_call(
        _epi2_body,
        grid=(n_blocks,),
        in_specs=[
            pl.BlockSpec((bn, H2 * C2), lambda i: (i, 0)),
            pl.BlockSpec((bn, H2 * C2), lambda i: (i, 0)),
            pl.BlockSpec((NC, bn, 80), lambda i: (0, i, 0)),
            pl.BlockSpec((NC, bn, 32), lambda i: (0, i, 0)),
            pl.BlockSpec(We2.shape, lambda i: (0, 0)),
            pl.BlockSpec((1, H2 * C2), lambda i: (0, 0)),
            pl.BlockSpec((1, H2 * C2), lambda i: (0, 0)),
            pl.BlockSpec((1, H2 * C2), lambda i: (0, 0)),
            pl.BlockSpec((1, H2 * C2), lambda i: (0, 0)),
        ],
        out_specs=pl.BlockSpec((bn, H2 * C2), lambda i: (i, 0)),
        out_shape=jax.ShapeDtypeStruct((n_pad, H2 * C2), f32),
    )(xl2[:n_pad], xr2[:n_pad], num2, asum, We2,
      row2d(att2.reshape(-1)), row2d(bias2), row2d(g2), row2d(b2))

    return out[:n_nodes]
